# baseline (speedup 1.0000x reference)
import functools

import jax
import jax.numpy as jnp
from jax.experimental import pallas as pl
from jax.experimental.pallas import tpu as pltpu

TABW = 256
HID = 16
PACK = 16
PACK1 = 64


def _ru(v, m):
  return (v + m - 1) // m * m


def _edge_messages(ea_ref, src_ref, dst_ref, tab_ref, w1_ref, b1_ref, w2_ref,
                   b2_ref, rb_ref, rr_ref, *, half, pack_shift):
  ea = ea_ref[...]
  e_tile = ea.shape[0]
  ng = tab_ref.shape[0]
  pack = 1 << pack_shift
  slot_shift = 8 - pack_shift

  eh = jnp.maximum(
      jnp.dot(ea, w1_ref[...], preferred_element_type=jnp.float32)
      + b1_ref[...], 0.0)
  ehh = eh[:, half * 64:(half + 1) * 64]
  we = jnp.dot(ehh, w2_ref[...], preferred_element_type=jnp.float32) + b2_ref[...]

  src = src_ref[...]
  src_hi = jax.lax.shift_right_arithmetic(src, pack_shift)
  src_lo = jax.lax.bitwise_and(src, pack - 1)
  oh = (jax.lax.broadcasted_iota(jnp.int32, (e_tile, ng), 1)
        == src_hi).astype(jnp.bfloat16)
  row = jnp.dot(oh, tab_ref[...], preferred_element_type=jnp.float32)

  lane = jax.lax.broadcasted_iota(jnp.int32, (e_tile, TABW), 1)
  sel = (jax.lax.shift_right_logical(lane, slot_shift) == src_lo)
  rxs = jnp.dot(row * sel.astype(jnp.float32), rb_ref[...],
                preferred_element_type=jnp.float32)

  rep = jnp.dot(rxs * we, rr_ref[...], preferred_element_type=jnp.float32)

  dst_lo = jax.lax.bitwise_and(dst_ref[...], PACK - 1)
  plc = (jax.lax.shift_right_logical(lane, 4) == dst_lo)
  return (rep * plc.astype(jnp.float32)).astype(jnp.bfloat16)


def _conv1_kernel(ea_ref, src_ref, dstc_ref, dstr_ref, tab_ref, w1_ref, b1_ref,
                  w2_ref, b2_ref, rb_ref, rr_ref, x16_ref, root_ref, bias_ref,
                  h_ref, rinv_ref, accm, accd, *, pack_shift):
  e_i = pl.program_id(0)

  @pl.when(e_i == 0)
  def _init():
    accm[...] = jnp.zeros_like(accm)
    accd[...] = jnp.zeros_like(accd)

  msg = _edge_messages(ea_ref, src_ref, dstc_ref, tab_ref, w1_ref, b1_ref,
                       w2_ref, b2_ref, rb_ref, rr_ref,
                       half=0, pack_shift=pack_shift)

  ng_tile = accm.shape[0]
  dstr = dstr_ref[...]
  e_tile = dstr.shape[1]
  dst_hi = jax.lax.shift_right_arithmetic(dstr, 4)
  oh = (jax.lax.broadcasted_iota(jnp.int32, (ng_tile, e_tile), 0)
        == dst_hi).astype(jnp.bfloat16)

  lane = jax.lax.broadcasted_iota(jnp.int32, (e_tile, TABW), 1)
  dst_lo = jax.lax.bitwise_and(dstc_ref[...], PACK - 1)
  ones = (jax.lax.shift_right_logical(lane, 4) == dst_lo).astype(jnp.bfloat16)
  cat = jnp.concatenate([msg, ones], axis=1)
  acc = jnp.dot(oh, cat, preferred_element_type=jnp.float32)
  accm[...] += acc[:, :TABW]
  accd[...] += acc[:, TABW:]

  @pl.when(e_i == pl.num_programs(0) - 1)
  def _finalize():
    deg = jnp.maximum(accd[...], 1.0)
    r = pl.reciprocal(deg, approx=True)
    r = r * (2.0 - deg * r)
    h = jnp.maximum(
        accm[...] * r
        + jnp.dot(x16_ref[...], root_ref[...],
                  preferred_element_type=jnp.float32)
        + bias_ref[...], 0.0)
    h_ref[...] = h
    rinv_ref[...] = r


def _conv1(ea, src_col, dst_col, dst_row, x4_bf16, w1c, b1c, w2, b2, rb, rr,
           x16, root_bd, bias_t, *, e_tile):
  ng = x16.shape[0]
  e_pad = ea.shape[0]
  args = [ea, src_col, dst_col, dst_row, x4_bf16, w1c, b1c, w2, b2, rb, rr,
          x16, root_bd, bias_t]
  flops = int(2 * e_pad * (x4_bf16.shape[0] * TABW + 2 * TABW * TABW
                           + 2 * ng * TABW) + 2 * ng * TABW * TABW)
  bytes_accessed = int(sum(int(a.size) * a.dtype.itemsize for a in args)
                       + 2 * ng * TABW * 4)
  return pl.pallas_call(
      functools.partial(_conv1_kernel, pack_shift=6),
      out_shape=(jax.ShapeDtypeStruct((ng, TABW), jnp.float32),
                 jax.ShapeDtypeStruct((ng, TABW), jnp.float32)),
      grid_spec=pltpu.PrefetchScalarGridSpec(
          num_scalar_prefetch=0,
          grid=(e_pad // e_tile,),
          in_specs=[
              pl.BlockSpec((e_tile, ea.shape[1]), lambda e: (e, 0)),
              pl.BlockSpec((e_tile, 1), lambda e: (e, 0)),
              pl.BlockSpec((e_tile, 1), lambda e: (e, 0)),
              pl.BlockSpec((1, e_tile), lambda e: (0, e)),
              pl.BlockSpec(x4_bf16.shape, lambda e: (0, 0)),
              pl.BlockSpec(w1c.shape, lambda e: (0, 0)),
              pl.BlockSpec(b1c.shape, lambda e: (0, 0)),
              pl.BlockSpec(w2.shape, lambda e: (0, 0)),
              pl.BlockSpec(b2.shape, lambda e: (0, 0)),
              pl.BlockSpec(rb.shape, lambda e: (0, 0)),
              pl.BlockSpec(rr.shape, lambda e: (0, 0)),
              pl.BlockSpec(x16.shape, lambda e: (0, 0)),
              pl.BlockSpec(root_bd.shape, lambda e: (0, 0)),
              pl.BlockSpec(bias_t.shape, lambda e: (0, 0)),
          ],
          out_specs=(pl.BlockSpec(x16.shape, lambda e: (0, 0)),
                     pl.BlockSpec(x16.shape, lambda e: (0, 0))),
          scratch_shapes=[pltpu.VMEM((ng, TABW), jnp.float32),
                          pltpu.VMEM((ng, TABW), jnp.float32)]),
      compiler_params=pltpu.CompilerParams(
          dimension_semantics=("arbitrary",)),
      cost_estimate=pl.CostEstimate(flops=flops, transcendentals=0,
                                    bytes_accessed=bytes_accessed),
  )(*args)


def _conv2_kernel(ea_ref, src_ref, dstc_ref, dstr_ref, tab_ref, w1_ref, b1_ref,
                  w2_ref, b2_ref, rb_ref, rr_ref, h16_ref, rinv_ref, root_ref,
                  bias_ref, fcw_ref, fcb_ref, out_ref, accm, *, pack_shift):
  e_i = pl.program_id(0)

  @pl.when(e_i == 0)
  def _init():
    accm[...] = jnp.zeros_like(accm)

  msg = _edge_messages(ea_ref, src_ref, dstc_ref, tab_ref, w1_ref, b1_ref,
                       w2_ref, b2_ref, rb_ref, rr_ref,
                       half=1, pack_shift=pack_shift)

  ng_tile = accm.shape[0]
  dstr = dstr_ref[...]
  e_tile = dstr.shape[1]
  dst_hi = jax.lax.shift_right_arithmetic(dstr, 4)
  oh = (jax.lax.broadcasted_iota(jnp.int32, (ng_tile, e_tile), 0)
        == dst_hi).astype(jnp.bfloat16)
  accm[...] += jnp.dot(oh, msg, preferred_element_type=jnp.float32)

  @pl.when(e_i == pl.num_programs(0) - 1)
  def _finalize():
    h = jnp.maximum(
        accm[...] * rinv_ref[...]
        + jnp.dot(h16_ref[...], root_ref[...],
                  preferred_element_type=jnp.float32)
        + bias_ref[...], 0.0)
    out_ref[...] = (jnp.dot(h, fcw_ref[...], preferred_element_type=jnp.float32)
                    + fcb_ref[...])


def _conv2(ea, src_col, dst_col, dst_row, h_bf16, w1c, b1c, w2, b2, rb, rr,
           h16, rinv, root_bd, bias_t, fcw_bd, fcb_t, *, e_tile):
  ng = h16.shape[0]
  e_pad = ea.shape[0]
  args = [ea, src_col, dst_col, dst_row, h_bf16, w1c, b1c, w2, b2, rb, rr,
          h16, rinv, root_bd, bias_t, fcw_bd, fcb_t]
  flops = int(2 * e_pad * (ng * TABW + 2 * TABW * TABW + ng * TABW)
              + 4 * ng * TABW * TABW)
  bytes_accessed = int(sum(int(a.size) * a.dtype.itemsize for a in args)
                       + ng * TABW * 4)
  return pl.pallas_call(
      functools.partial(_conv2_kernel, pack_shift=4),
      out_shape=jax.ShapeDtypeStruct((ng, TABW), jnp.float32),
      grid_spec=pltpu.PrefetchScalarGridSpec(
          num_scalar_prefetch=0,
          grid=(e_pad // e_tile,),
          in_specs=[
              pl.BlockSpec((e_tile, ea.shape[1]), lambda e: (e, 0)),
              pl.BlockSpec((e_tile, 1), lambda e: (e, 0)),
              pl.BlockSpec((e_tile, 1), lambda e: (e, 0)),
              pl.BlockSpec((1, e_tile), lambda e: (0, e)),
              pl.BlockSpec(h_bf16.shape, lambda e: (0, 0)),
              pl.BlockSpec(w1c.shape, lambda e: (0, 0)),
              pl.BlockSpec(b1c.shape, lambda e: (0, 0)),
              pl.BlockSpec(w2.shape, lambda e: (0, 0)),
              pl.BlockSpec(b2.shape, lambda e: (0, 0)),
              pl.BlockSpec(rb.shape, lambda e: (0, 0)),
              pl.BlockSpec(rr.shape, lambda e: (0, 0)),
              pl.BlockSpec(h16.shape, lambda e: (0, 0)),
              pl.BlockSpec(rinv.shape, lambda e: (0, 0)),
              pl.BlockSpec(root_bd.shape, lambda e: (0, 0)),
              pl.BlockSpec(bias_t.shape, lambda e: (0, 0)),
              pl.BlockSpec(fcw_bd.shape, lambda e: (0, 0)),
              pl.BlockSpec(fcb_t.shape, lambda e: (0, 0)),
          ],
          out_specs=pl.BlockSpec(h16.shape, lambda e: (0, 0)),
          scratch_shapes=[pltpu.VMEM((ng, TABW), jnp.float32)]),
      compiler_params=pltpu.CompilerParams(
          dimension_semantics=("arbitrary",)),
      cost_estimate=pl.CostEstimate(flops=flops, transcendentals=0,
                                    bytes_accessed=bytes_accessed),
  )(*args)


def kernel(x, edge_index, edge_attr, e1w1, e1b1, e1w2, e1b2, root1, bias1,
           e2w1, e2b1, e2w2, e2b2, root2, bias2, fcw, fcb):
  n_nodes, node_in = x.shape
  n_edges = edge_attr.shape[0]
  hid = root1.shape[1]
  out_dim = fcw.shape[1]

  e_tile = 4096
  n_pad = _ru(max(n_nodes, 4096), PACK1 * 8)
  e_pad = _ru(max(n_edges, e_tile), e_tile)
  ng = n_pad // PACK
  ng1 = n_pad // PACK1

  f32 = jnp.float32
  x4 = jnp.zeros((n_pad, 4), f32).at[:n_nodes, :node_in].set(x)
  x4 = x4.reshape(ng1, TABW)
  x16 = jnp.zeros((n_pad, HID), f32).at[:n_nodes, :node_in].set(x)
  x16 = x16.reshape(ng, TABW)

  ea = jnp.zeros((e_pad, edge_attr.shape[1]), f32).at[:n_edges].set(edge_attr)
  src_col = jnp.full((e_pad, 1), -1, jnp.int32).at[:n_edges, 0].set(
      edge_index[0].astype(jnp.int32))
  dst_col = jnp.full((e_pad, 1), -1, jnp.int32).at[:n_edges, 0].set(
      edge_index[1].astype(jnp.int32))
  dst_row = dst_col.reshape(1, e_pad)

  w1c = jnp.concatenate([e1w1, e2w1], axis=1)
  b1c = jnp.concatenate([e1b1, e2b1], axis=1)

  ones16 = jnp.ones((1, HID), f32)
  rb1 = jnp.tile(jnp.kron(jnp.eye(node_in, dtype=f32), ones16), (PACK1, 1))
  rb2 = jnp.tile(jnp.kron(jnp.eye(HID, dtype=f32), ones16), (PACK, 1))
  t16 = jnp.tile(jnp.eye(HID, dtype=f32), (1, PACK))
  rr1 = jnp.tile(t16, (node_in, 1))
  rr2 = jnp.tile(t16, (HID, 1))

  eyep = jnp.eye(PACK, dtype=f32)
  root1_bd = jnp.kron(eyep, jnp.pad(root1, ((0, HID - node_in), (0, 0))))
  root2_bd = jnp.kron(eyep, root2)
  fcw_bd = jnp.kron(eyep, jnp.pad(fcw, ((0, 0), (0, HID - out_dim))))
  bias1_t = jnp.tile(bias1, (1, PACK))
  bias2_t = jnp.tile(bias2, (1, PACK))
  fcb_t = jnp.tile(jnp.pad(fcb, ((0, 0), (0, HID - out_dim))), (1, PACK))

  h1, rinv = _conv1(ea, src_col, dst_col, dst_row, x4.astype(jnp.bfloat16),
                    w1c, b1c, e1w2, e1b2, rb1, rr1, x16, root1_bd, bias1_t,
                    e_tile=e_tile)

  out = _conv2(ea, src_col, dst_col, dst_row, h1.astype(jnp.bfloat16),
               w1c, b1c, e2w2, e2b2, rb2, rr2, h1, rinv,
               root2_bd, bias2_t, fcw_bd, fcb_t, e_tile=e_tile)

  return out.reshape(n_pad, HID)[:n_nodes, :out_dim]


# speedup vs baseline: 1.1221x; 1.1221x over previous
import functools

import jax
import jax.numpy as jnp
from jax.experimental import pallas as pl
from jax.experimental.pallas import tpu as pltpu

TABW = 256
HID = 16
PACK = 16
PACK1 = 64


def _ru(v, m):
  return (v + m - 1) // m * m


def _msg_kernel(ea_ref, src_ref, dst_ref, tab_ref, w1_ref, b1_ref, w2_ref,
                b2_ref, rb_ref, rr_ref, out_ref, *, half, pack_shift):
  ea = ea_ref[...]
  e_tile = ea.shape[0]
  ng = tab_ref.shape[0]
  pack = 1 << pack_shift
  slot_shift = 8 - pack_shift

  eh = jnp.maximum(
      jnp.dot(ea, w1_ref[...], preferred_element_type=jnp.float32)
      + b1_ref[...], 0.0)
  ehh = eh[:, half * 64:(half + 1) * 64]
  we = jnp.dot(ehh, w2_ref[...], preferred_element_type=jnp.float32) + b2_ref[...]

  src = src_ref[...]
  src_hi = jax.lax.shift_right_arithmetic(src, pack_shift)
  src_lo = jax.lax.bitwise_and(src, pack - 1)
  oh = (jax.lax.broadcasted_iota(jnp.int32, (e_tile, ng), 1)
        == src_hi).astype(jnp.bfloat16)
  row = jnp.dot(oh, tab_ref[...], preferred_element_type=jnp.float32)

  lane = jax.lax.broadcasted_iota(jnp.int32, (e_tile, TABW), 1)
  sel = (jax.lax.shift_right_logical(lane, slot_shift) == src_lo)
  rxs = jnp.dot(row * sel.astype(jnp.float32), rb_ref[...],
                preferred_element_type=jnp.float32)

  rep = jnp.dot(rxs * we, rr_ref[...], preferred_element_type=jnp.float32)

  dst_lo = jax.lax.bitwise_and(dst_ref[...], PACK - 1)
  plc = (jax.lax.shift_right_logical(lane, 4) == dst_lo)
  out_ref[...] = (rep * plc.astype(jnp.float32)).astype(jnp.bfloat16)


def _messages(ea, src_col, dst_col, tab_bf16, w1c, b1c, w2, b2, rb, rr, *,
              half, pack_shift, e_tile):
  e_pad = ea.shape[0]
  ng = tab_bf16.shape[0]
  args = [ea, src_col, dst_col, tab_bf16, w1c, b1c, w2, b2, rb, rr]
  flops = int(2 * e_pad * (ng * TABW + 64 * w2.shape[1] + 2 * TABW * TABW))
  bytes_accessed = int(sum(int(a.size) * a.dtype.itemsize for a in args)
                       + e_pad * TABW * 2)
  return pl.pallas_call(
      functools.partial(_msg_kernel, half=half, pack_shift=pack_shift),
      out_shape=jax.ShapeDtypeStruct((e_pad, TABW), jnp.bfloat16),
      grid_spec=pltpu.PrefetchScalarGridSpec(
          num_scalar_prefetch=0,
          grid=(e_pad // e_tile,),
          in_specs=[
              pl.BlockSpec((e_tile, ea.shape[1]), lambda e: (e, 0)),
              pl.BlockSpec((e_tile, 1), lambda e: (e, 0)),
              pl.BlockSpec((e_tile, 1), lambda e: (e, 0)),
              pl.BlockSpec(tab_bf16.shape, lambda e: (0, 0)),
              pl.BlockSpec(w1c.shape, lambda e: (0, 0)),
              pl.BlockSpec(b1c.shape, lambda e: (0, 0)),
              pl.BlockSpec(w2.shape, lambda e: (0, 0)),
              pl.BlockSpec(b2.shape, lambda e: (0, 0)),
              pl.BlockSpec(rb.shape, lambda e: (0, 0)),
              pl.BlockSpec(rr.shape, lambda e: (0, 0)),
          ],
          out_specs=pl.BlockSpec((e_tile, TABW), lambda e: (e, 0))),
      compiler_params=pltpu.CompilerParams(dimension_semantics=("parallel",)),
      cost_estimate=pl.CostEstimate(flops=flops, transcendentals=0,
                                    bytes_accessed=bytes_accessed),
  )(*args)


def _scatter1_kernel(msg_ref, dstr_ref, dstc_ref, x16_ref, root_ref, bias_ref,
                     h_ref, rinv_ref, accm, accd):
  e_i = pl.program_id(1)

  @pl.when(e_i == 0)
  def _init():
    accm[...] = jnp.zeros_like(accm)
    accd[...] = jnp.zeros_like(accd)

  ng_tile = accm.shape[0]
  dstr = dstr_ref[...]
  e_tile = dstr.shape[1]
  base = pl.program_id(0) * ng_tile
  dst_hi = jax.lax.shift_right_arithmetic(dstr, 4)
  oh = ((jax.lax.broadcasted_iota(jnp.int32, (ng_tile, e_tile), 0) + base)
        == dst_hi).astype(jnp.bfloat16)

  lane = jax.lax.broadcasted_iota(jnp.int32, (e_tile, TABW), 1)
  dst_lo = jax.lax.bitwise_and(dstc_ref[...], PACK - 1)
  ones = (jax.lax.shift_right_logical(lane, 4) == dst_lo).astype(jnp.bfloat16)
  cat = jnp.concatenate([msg_ref[...], ones], axis=1)
  acc = jnp.dot(oh, cat, preferred_element_type=jnp.float32)
  accm[...] += acc[:, :TABW]
  accd[...] += acc[:, TABW:]

  @pl.when(e_i == pl.num_programs(1) - 1)
  def _finalize():
    deg = jnp.maximum(accd[...], 1.0)
    r = pl.reciprocal(deg, approx=True)
    r = r * (2.0 - deg * r)
    h = jnp.maximum(
        accm[...] * r
        + jnp.dot(x16_ref[...], root_ref[...],
                  preferred_element_type=jnp.float32)
        + bias_ref[...], 0.0)
    h_ref[...] = h
    rinv_ref[...] = r


def _scatter1(msgs, dst_row, dst_col, x16, root_bd, bias_t, *, ng_tile, e_tile):
  ng = x16.shape[0]
  e_pad = msgs.shape[0]
  args = [msgs, dst_row, dst_col, x16, root_bd, bias_t]
  flops = int(2 * ng * e_pad * 2 * TABW + 2 * ng * TABW * TABW)
  bytes_accessed = int((ng // ng_tile) * e_pad * TABW * 2
                       + 2 * ng * TABW * 4 + x16.size * 4)
  return pl.pallas_call(
      _scatter1_kernel,
      out_shape=(jax.ShapeDtypeStruct((ng, TABW), jnp.float32),
                 jax.ShapeDtypeStruct((ng, TABW), jnp.float32)),
      grid_spec=pltpu.PrefetchScalarGridSpec(
          num_scalar_prefetch=0,
          grid=(ng // ng_tile, e_pad // e_tile),
          in_specs=[
              pl.BlockSpec((e_tile, TABW), lambda n, e: (e, 0)),
              pl.BlockSpec((1, e_tile), lambda n, e: (0, e)),
              pl.BlockSpec((e_tile, 1), lambda n, e: (e, 0)),
              pl.BlockSpec((ng_tile, TABW), lambda n, e: (n, 0)),
              pl.BlockSpec(root_bd.shape, lambda n, e: (0, 0)),
              pl.BlockSpec(bias_t.shape, lambda n, e: (0, 0)),
          ],
          out_specs=(pl.BlockSpec((ng_tile, TABW), lambda n, e: (n, 0)),
                     pl.BlockSpec((ng_tile, TABW), lambda n, e: (n, 0))),
          scratch_shapes=[pltpu.VMEM((ng_tile, TABW), jnp.float32),
                          pltpu.VMEM((ng_tile, TABW), jnp.float32)]),
      compiler_params=pltpu.CompilerParams(
          dimension_semantics=("parallel", "arbitrary")),
      cost_estimate=pl.CostEstimate(flops=flops, transcendentals=0,
                                    bytes_accessed=bytes_accessed),
  )(*args)


def _scatter2_kernel(msg_ref, dstr_ref, h16_ref, rinv_ref, root_ref, bias_ref,
                     fcw_ref, fcb_ref, out_ref, accm):
  e_i = pl.program_id(1)

  @pl.when(e_i == 0)
  def _init():
    accm[...] = jnp.zeros_like(accm)

  ng_tile = accm.shape[0]
  dstr = dstr_ref[...]
  e_tile = dstr.shape[1]
  base = pl.program_id(0) * ng_tile
  dst_hi = jax.lax.shift_right_arithmetic(dstr, 4)
  oh = ((jax.lax.broadcasted_iota(jnp.int32, (ng_tile, e_tile), 0) + base)
        == dst_hi).astype(jnp.bfloat16)
  accm[...] += jnp.dot(oh, msg_ref[...], preferred_element_type=jnp.float32)

  @pl.when(e_i == pl.num_programs(1) - 1)
  def _finalize():
    h = jnp.maximum(
        accm[...] * rinv_ref[...]
        + jnp.dot(h16_ref[...], root_ref[...],
                  preferred_element_type=jnp.float32)
        + bias_ref[...], 0.0)
    out_ref[...] = (jnp.dot(h, fcw_ref[...], preferred_element_type=jnp.float32)
                    + fcb_ref[...])


def _scatter2(msgs, dst_row, h16, rinv, root_bd, bias_t, fcw_bd, fcb_t, *,
              ng_tile, e_tile):
  ng = h16.shape[0]
  e_pad = msgs.shape[0]
  args = [msgs, dst_row, h16, rinv, root_bd, bias_t, fcw_bd, fcb_t]
  flops = int(2 * ng * e_pad * TABW + 4 * ng * TABW * TABW)
  bytes_accessed = int((ng // ng_tile) * e_pad * TABW * 2
                       + 3 * ng * TABW * 4)
  return pl.pallas_call(
      _scatter2_kernel,
      out_shape=jax.ShapeDtypeStruct((ng, TABW), jnp.float32),
      grid_spec=pltpu.PrefetchScalarGridSpec(
          num_scalar_prefetch=0,
          grid=(ng // ng_tile, e_pad // e_tile),
          in_specs=[
              pl.BlockSpec((e_tile, TABW), lambda n, e: (e, 0)),
              pl.BlockSpec((1, e_tile), lambda n, e: (0, e)),
              pl.BlockSpec((ng_tile, TABW), lambda n, e: (n, 0)),
              pl.BlockSpec((ng_tile, TABW), lambda n, e: (n, 0)),
              pl.BlockSpec(root_bd.shape, lambda n, e: (0, 0)),
              pl.BlockSpec(bias_t.shape, lambda n, e: (0, 0)),
              pl.BlockSpec(fcw_bd.shape, lambda n, e: (0, 0)),
              pl.BlockSpec(fcb_t.shape, lambda n, e: (0, 0)),
          ],
          out_specs=pl.BlockSpec((ng_tile, TABW), lambda n, e: (n, 0)),
          scratch_shapes=[pltpu.VMEM((ng_tile, TABW), jnp.float32)]),
      compiler_params=pltpu.CompilerParams(
          dimension_semantics=("parallel", "arbitrary")),
      cost_estimate=pl.CostEstimate(flops=flops, transcendentals=0,
                                    bytes_accessed=bytes_accessed),
  )(*args)


def kernel(x, edge_index, edge_attr, e1w1, e1b1, e1w2, e1b2, root1, bias1,
           e2w1, e2b1, e2w2, e2b2, root2, bias2, fcw, fcb):
  n_nodes, node_in = x.shape
  n_edges = edge_attr.shape[0]
  hid = root1.shape[1]
  out_dim = fcw.shape[1]

  e_tile = 4096
  n_pad = _ru(max(n_nodes, 4096), PACK1 * 8)
  e_pad = _ru(max(n_edges, e_tile), e_tile)
  ng = n_pad // PACK
  ng1 = n_pad // PACK1
  ng_tile = ng
  for cand in (1024, 512, 256, 128, 64):
    if ng % cand == 0:
      ng_tile = cand
      break

  f32 = jnp.float32
  x4 = jnp.zeros((n_pad, 4), f32).at[:n_nodes, :node_in].set(x)
  x4 = x4.reshape(ng1, TABW)
  x16 = jnp.zeros((n_pad, HID), f32).at[:n_nodes, :node_in].set(x)
  x16 = x16.reshape(ng, TABW)

  ea = jnp.zeros((e_pad, edge_attr.shape[1]), f32).at[:n_edges].set(edge_attr)
  src_col = jnp.full((e_pad, 1), -1, jnp.int32).at[:n_edges, 0].set(
      edge_index[0].astype(jnp.int32))
  dst_col = jnp.full((e_pad, 1), -1, jnp.int32).at[:n_edges, 0].set(
      edge_index[1].astype(jnp.int32))
  dst_row = dst_col.reshape(1, e_pad)

  w1c = jnp.concatenate([e1w1, e2w1], axis=1)
  b1c = jnp.concatenate([e1b1, e2b1], axis=1)

  ones16 = jnp.ones((1, HID), f32)
  rb1 = jnp.tile(jnp.kron(jnp.eye(node_in, dtype=f32), ones16), (PACK1, 1))
  rb2 = jnp.tile(jnp.kron(jnp.eye(HID, dtype=f32), ones16), (PACK, 1))
  t16 = jnp.tile(jnp.eye(HID, dtype=f32), (1, PACK))
  rr1 = jnp.tile(t16, (node_in, 1))
  rr2 = jnp.tile(t16, (HID, 1))

  eyep = jnp.eye(PACK, dtype=f32)
  root1_bd = jnp.kron(eyep, jnp.pad(root1, ((0, HID - node_in), (0, 0))))
  root2_bd = jnp.kron(eyep, root2)
  fcw_bd = jnp.kron(eyep, jnp.pad(fcw, ((0, 0), (0, HID - out_dim))))
  bias1_t = jnp.tile(bias1, (1, PACK))
  bias2_t = jnp.tile(bias2, (1, PACK))
  fcb_t = jnp.tile(jnp.pad(fcb, ((0, 0), (0, HID - out_dim))), (1, PACK))

  msgs1 = _messages(ea, src_col, dst_col, x4.astype(jnp.bfloat16),
                    w1c, b1c, e1w2, e1b2, rb1, rr1,
                    half=0, pack_shift=6, e_tile=e_tile)
  h1, rinv = _scatter1(msgs1, dst_row, dst_col, x16, root1_bd, bias1_t,
                       ng_tile=ng_tile, e_tile=e_tile)

  msgs2 = _messages(ea, src_col, dst_col, h1.astype(jnp.bfloat16),
                    w1c, b1c, e2w2, e2b2, rb2, rr2,
                    half=1, pack_shift=4, e_tile=e_tile)
  out = _scatter2(msgs2, dst_row, h1, rinv, root2_bd, bias2_t, fcw_bd, fcb_t,
                  ng_tile=ng_tile, e_tile=e_tile)

  return out.reshape(n_pad, HID)[:n_nodes, :out_dim]
